# Initial kernel scaffold; baseline (speedup 1.0000x reference)
import jax
import jax.numpy as jnp
from jax.experimental import pallas as pl


def kernel(x, child_idx, W_iou_w, W_iou_b, U_iou_w, U_iou_b, U_f_w, U_f_b):
    raise NotImplementedError("write your pallas kernel here")


# speedup vs baseline: 2.0225x; 2.0225x over previous
import functools

import jax
import jax.numpy as jnp
from jax import lax
from jax.experimental import pallas as pl
from jax.experimental.pallas import tpu as pltpu
from jax.experimental.pallas import tpu_sc as plsc

N, M, X_SIZE, H_SIZE = 100000, 50000, 128, 128
H2, H3 = 2 * H_SIZE, 3 * H_SIZE

_NC, _NS = 2, 16
_NW = _NC * _NS

_MP = ((M + 8 * _NW - 1) // (8 * _NW)) * (8 * _NW)
_R = 2 * _MP
_R_PW = _R // _NW
_CH = 64
_NCHUNK = _R_PW // _CH

_BN = 2000
_BM = 2000

_PREC = lax.Precision.HIGHEST


def _leaf_body(x_ref, w_ref, b_ref, out_ref):
    iou = jnp.dot(x_ref[...], w_ref[...], precision=_PREC,
                  preferred_element_type=jnp.float32) + b_ref[...]
    i = jax.nn.sigmoid(iou[:, :H_SIZE])
    o = jax.nn.sigmoid(iou[:, H_SIZE:H2])
    u = jnp.tanh(iou[:, H2:])
    c0 = i * u
    out_ref[:, :H_SIZE] = o * jnp.tanh(c0)
    out_ref[:, H_SIZE:] = c0


def _leaf_update(x, w_t, b):
    return pl.pallas_call(
        _leaf_body,
        grid=(N // _BN,),
        in_specs=[
            pl.BlockSpec((_BN, X_SIZE), lambda j: (j, 0)),
            pl.BlockSpec((X_SIZE, H3), lambda j: (0, 0)),
            pl.BlockSpec((1, H3), lambda j: (0, 0)),
        ],
        out_specs=pl.BlockSpec((_BN, H2), lambda j: (j, 0)),
        out_shape=jax.ShapeDtypeStruct((N, H2), jnp.float32),
    )(x, w_t, b)


def _sc_gather_body(table_hbm, idx_hbm, out_hbm, idx_v, rows_v, sem):
    wid = lax.axis_index("s") * _NC + lax.axis_index("c")
    base = wid * _R_PW
    pltpu.sync_copy(idx_hbm.at[pl.ds(base, _R_PW)], idx_v)

    def chunk(k, carry):
        off = k * _CH
        pltpu.async_copy(table_hbm.at[idx_v.at[pl.ds(off, _CH)]],
                         rows_v, sem).wait()
        pltpu.sync_copy(rows_v, out_hbm.at[pl.ds(base + off, _CH)])
        return carry

    lax.fori_loop(0, _NCHUNK, chunk, 0)


@functools.cache
def _sc_gather_kernel():
    return pl.kernel(
        _sc_gather_body,
        out_type=jax.ShapeDtypeStruct((_R, H2), jnp.float32),
        mesh=plsc.VectorSubcoreMesh(core_axis_name="c", subcore_axis_name="s",
                                    num_cores=_NC, num_subcores=_NS),
        scratch_types=[
            pltpu.VMEM((_R_PW,), jnp.int32),
            pltpu.VMEM((_CH, H2), jnp.float32),
            pltpu.SemaphoreType.DMA,
        ],
    )


def _update_body(g_ref, w_ref, b_ref, out_ref):
    g0 = g_ref[0]
    g1 = g_ref[1]
    h0c, c0c = g0[:, :H_SIZE], g0[:, H_SIZE:]
    h1c, c1c = g1[:, :H_SIZE], g1[:, H_SIZE:]
    z = (jnp.dot(h0c, w_ref[:H_SIZE, :], precision=_PREC,
                 preferred_element_type=jnp.float32)
         + jnp.dot(h1c, w_ref[H_SIZE:, :], precision=_PREC,
                   preferred_element_type=jnp.float32)
         + b_ref[...])
    f0 = jax.nn.sigmoid(z[:, :H_SIZE])
    f1 = jax.nn.sigmoid(z[:, H_SIZE:H2])
    i = jax.nn.sigmoid(z[:, H2:H2 + H_SIZE])
    o = jax.nn.sigmoid(z[:, H2 + H_SIZE:2 * H2])
    u = jnp.tanh(z[:, 2 * H2:])
    c = i * u + f0 * c0c + f1 * c1c
    out_ref[:, :H_SIZE] = o * jnp.tanh(c)
    out_ref[:, H_SIZE:] = c


def _update(g3, w_cat, b_cat):
    return pl.pallas_call(
        _update_body,
        grid=(M // _BM,),
        in_specs=[
            pl.BlockSpec((2, _BM, H2), lambda j: (0, j, 0)),
            pl.BlockSpec((H2, H2 + H3), lambda j: (0, 0)),
            pl.BlockSpec((1, H2 + H3), lambda j: (0, 0)),
        ],
        out_specs=pl.BlockSpec((_BM, H2), lambda j: (j, 0)),
        out_shape=jax.ShapeDtypeStruct((M, H2), jnp.float32),
    )(g3, w_cat, b_cat)


def kernel(x, child_idx, W_iou_w, W_iou_b, U_iou_w, U_iou_b, U_f_w, U_f_b):
    w_iou_t = W_iou_w.T
    b_iou = W_iou_b.reshape(1, H3)
    w_cat = jnp.concatenate([U_f_w.T, U_iou_w.T], axis=1)
    b_cat = jnp.concatenate([U_f_b, U_iou_b]).reshape(1, H2 + H3)

    hc = _leaf_update(x, w_iou_t, b_iou)

    pad = _MP - M
    idx0 = jnp.pad(child_idx[:, 0], (0, pad))
    idx1 = jnp.pad(child_idx[:, 1], (0, pad))
    idx_flat = jnp.concatenate([idx0, idx1]).astype(jnp.int32)

    g = _sc_gather_kernel()(hc, idx_flat)
    g3 = g.reshape(2, _MP, H2)

    return _update(g3, w_cat, b_cat)


# speedup vs baseline: 2.5553x; 1.2634x over previous
import functools

import jax
import jax.numpy as jnp
from jax import lax
from jax.experimental import pallas as pl
from jax.experimental.pallas import tpu as pltpu
from jax.experimental.pallas import tpu_sc as plsc

N, M, X_SIZE, H_SIZE = 100000, 50000, 128, 128
H2, H3 = 2 * H_SIZE, 3 * H_SIZE

_NC, _NS = 2, 16
_NW = _NC * _NS

_MP = 51200
_R = 2 * _MP
_R_PW = _R // _NW
_CH = 64
_NCHUNK = _R_PW // _CH

_BN = 2000
_BM = 2000

_PREC = lax.Precision.HIGHEST


def _leaf_body(x_ref, w_ref, b_ref, out_ref):
    iou = jnp.dot(x_ref[...], w_ref[...], precision=_PREC,
                  preferred_element_type=jnp.float32) + b_ref[...]
    i = jax.nn.sigmoid(iou[:, :H_SIZE])
    o = jax.nn.sigmoid(iou[:, H_SIZE:H2])
    u = jnp.tanh(iou[:, H2:])
    c0 = i * u
    h0 = o * jnp.tanh(c0)
    hb = lax.bitcast_convert_type(h0.astype(jnp.bfloat16),
                                  jnp.uint16).astype(jnp.uint32)
    cb = lax.bitcast_convert_type(c0.astype(jnp.bfloat16),
                                  jnp.uint16).astype(jnp.uint32)
    out_ref[...] = lax.bitcast_convert_type((hb << 16) | cb, jnp.float32)


def _leaf_update(x, w_t, b):
    return pl.pallas_call(
        _leaf_body,
        grid=(N // _BN,),
        in_specs=[
            pl.BlockSpec((_BN, X_SIZE), lambda j: (j, 0)),
            pl.BlockSpec((X_SIZE, H3), lambda j: (0, 0)),
            pl.BlockSpec((1, H3), lambda j: (0, 0)),
        ],
        out_specs=pl.BlockSpec((_BN, H_SIZE), lambda j: (j, 0)),
        out_shape=jax.ShapeDtypeStruct((N, H_SIZE), jnp.float32),
    )(x, w_t, b)


def _sc_gather_body(table_hbm, idx_hbm, out_hbm, idx_v, rows_v, sem):
    wid = lax.axis_index("s") * _NC + lax.axis_index("c")
    base = wid * _R_PW
    pltpu.sync_copy(idx_hbm.at[pl.ds(base, _R_PW)], idx_v)

    def chunk(k, carry):
        off = k * _CH
        pltpu.async_copy(table_hbm.at[idx_v.at[pl.ds(off, _CH)]],
                         rows_v, sem).wait()
        pltpu.sync_copy(rows_v, out_hbm.at[pl.ds(base + off, _CH)])
        return carry

    lax.fori_loop(0, _NCHUNK, chunk, 0)


@functools.cache
def _sc_gather_kernel():
    return pl.kernel(
        _sc_gather_body,
        out_type=jax.ShapeDtypeStruct((_R, H_SIZE), jnp.float32),
        mesh=plsc.VectorSubcoreMesh(core_axis_name="c", subcore_axis_name="s",
                                    num_cores=_NC, num_subcores=_NS),
        scratch_types=[
            pltpu.VMEM((_R_PW,), jnp.int32),
            pltpu.VMEM((_CH, H_SIZE), jnp.float32),
            pltpu.SemaphoreType.DMA,
        ],
    )


def _unpack_hc(packed_f32):
    w = lax.bitcast_convert_type(packed_f32, jnp.uint32)
    h = lax.bitcast_convert_type((w >> 16).astype(jnp.uint16), jnp.bfloat16)
    c = lax.bitcast_convert_type(w.astype(jnp.uint16), jnp.bfloat16)
    return h, c.astype(jnp.float32)


def _update_body(g_ref, w_ref, b_ref, out_ref):
    h0c, c0c = _unpack_hc(g_ref[0])
    h1c, c1c = _unpack_hc(g_ref[1])
    z = (jnp.dot(h0c, w_ref[:H_SIZE, :],
                 preferred_element_type=jnp.float32)
         + jnp.dot(h1c, w_ref[H_SIZE:, :],
                   preferred_element_type=jnp.float32)
         + b_ref[...])
    f0 = jax.nn.sigmoid(z[:, :H_SIZE])
    f1 = jax.nn.sigmoid(z[:, H_SIZE:H2])
    i = jax.nn.sigmoid(z[:, H2:H2 + H_SIZE])
    o = jax.nn.sigmoid(z[:, H2 + H_SIZE:2 * H2])
    u = jnp.tanh(z[:, 2 * H2:])
    c = i * u + f0 * c0c + f1 * c1c
    out_ref[:, :H_SIZE] = o * jnp.tanh(c)
    out_ref[:, H_SIZE:] = c


def _update(g3, w_cat, b_cat):
    return pl.pallas_call(
        _update_body,
        grid=(M // _BM,),
        in_specs=[
            pl.BlockSpec((2, _BM, H_SIZE), lambda j: (0, j, 0)),
            pl.BlockSpec((H2, H2 + H3), lambda j: (0, 0)),
            pl.BlockSpec((1, H2 + H3), lambda j: (0, 0)),
        ],
        out_specs=pl.BlockSpec((_BM, H2), lambda j: (j, 0)),
        out_shape=jax.ShapeDtypeStruct((M, H2), jnp.float32),
    )(g3, w_cat, b_cat)


def kernel(x, child_idx, W_iou_w, W_iou_b, U_iou_w, U_iou_b, U_f_w, U_f_b):
    w_iou_t = W_iou_w.T
    b_iou = W_iou_b.reshape(1, H3)
    w_cat = jnp.concatenate([U_f_w.T, U_iou_w.T],
                            axis=1).astype(jnp.bfloat16)
    b_cat = jnp.concatenate([U_f_b, U_iou_b]).reshape(1, H2 + H3)

    hc = _leaf_update(x, w_iou_t, b_iou)

    pad = _MP - M
    idx0 = jnp.pad(child_idx[:, 0], (0, pad))
    idx1 = jnp.pad(child_idx[:, 1], (0, pad))
    idx_flat = jnp.concatenate([idx0, idx1]).astype(jnp.int32)

    g = _sc_gather_kernel()(hc, idx_flat)
    g3 = g.reshape(2, _MP, H_SIZE)

    return _update(g3, w_cat, b_cat)


# speedup vs baseline: 2.7977x; 1.0949x over previous
import functools

import jax
import jax.numpy as jnp
from jax import lax
from jax.experimental import pallas as pl
from jax.experimental.pallas import tpu as pltpu
from jax.experimental.pallas import tpu_sc as plsc

N, M, X_SIZE, H_SIZE = 100000, 50000, 128, 128
H2, H3 = 2 * H_SIZE, 3 * H_SIZE

_NC, _NS = 2, 16
_NW = _NC * _NS

_MP = 51200
_R = 2 * _MP
_R_PW = _R // _NW
_CH = 80
_NCHUNK = _R_PW // _CH
_NBUF = 4
_NGRP = _NCHUNK // _NBUF

_BN = 2000
_BM = 2000

_PREC = lax.Precision.HIGHEST


def _leaf_body(x_ref, w_ref, b_ref, out_ref):
    iou = jnp.dot(x_ref[...], w_ref[...], precision=_PREC,
                  preferred_element_type=jnp.float32) + b_ref[...]
    i = jax.nn.sigmoid(iou[:, :H_SIZE])
    o = jax.nn.sigmoid(iou[:, H_SIZE:H2])
    u = jnp.tanh(iou[:, H2:])
    c0 = i * u
    h0 = o * jnp.tanh(c0)
    hb = lax.bitcast_convert_type(h0.astype(jnp.bfloat16),
                                  jnp.uint16).astype(jnp.uint32)
    cb = lax.bitcast_convert_type(c0.astype(jnp.bfloat16),
                                  jnp.uint16).astype(jnp.uint32)
    out_ref[...] = lax.bitcast_convert_type((hb << 16) | cb, jnp.float32)


def _leaf_update(x, w_t, b):
    return pl.pallas_call(
        _leaf_body,
        grid=(N // _BN,),
        in_specs=[
            pl.BlockSpec((_BN, X_SIZE), lambda j: (j, 0)),
            pl.BlockSpec((X_SIZE, H3), lambda j: (0, 0)),
            pl.BlockSpec((1, H3), lambda j: (0, 0)),
        ],
        out_specs=pl.BlockSpec((_BN, H_SIZE), lambda j: (j, 0)),
        out_shape=jax.ShapeDtypeStruct((N, H_SIZE), jnp.float32),
    )(x, w_t, b)


def _sc_gather_body(table_hbm, idx_hbm, out_hbm, idx_v, rows_v, gsems, wsems):
    wid = lax.axis_index("s") * _NC + lax.axis_index("c")
    base = wid * _R_PW
    pltpu.sync_copy(idx_hbm.at[pl.ds(base, _R_PW)], idx_v)

    def fire(c, b):
        pltpu.async_copy(table_hbm.at[idx_v.at[pl.ds(c * _CH, _CH)]],
                         rows_v.at[b], gsems.at[b])

    def drain_write(c, b):
        pltpu.make_async_copy(table_hbm.at[idx_v.at[pl.ds(c * _CH, _CH)]],
                              rows_v.at[b], gsems.at[b]).wait()
        cp = pltpu.make_async_copy(rows_v.at[b],
                                   out_hbm.at[pl.ds(base + c * _CH, _CH)],
                                   wsems.at[b])
        cp.start()
        cp.wait()

    for b in range(_NBUF):
        fire(b, b)

    def group(j, carry):
        for b in range(_NBUF):
            c = j * _NBUF + b
            drain_write(c, b)
            fire(c + _NBUF, b)
        return carry

    lax.fori_loop(0, _NGRP - 1, group, 0)
    for b in range(_NBUF):
        drain_write((_NGRP - 1) * _NBUF + b, b)


@functools.cache
def _sc_gather_kernel():
    return pl.kernel(
        _sc_gather_body,
        out_type=jax.ShapeDtypeStruct((_R, H_SIZE), jnp.float32),
        mesh=plsc.VectorSubcoreMesh(core_axis_name="c", subcore_axis_name="s",
                                    num_cores=_NC, num_subcores=_NS),
        scratch_types=[
            pltpu.VMEM((_R_PW,), jnp.int32),
            pltpu.VMEM((_NBUF, _CH, H_SIZE), jnp.float32),
            pltpu.SemaphoreType.DMA((_NBUF,)),
            pltpu.SemaphoreType.DMA((_NBUF,)),
        ],
    )


def _unpack_hc(packed_f32):
    w = lax.bitcast_convert_type(packed_f32, jnp.uint32)
    h = lax.bitcast_convert_type((w >> 16).astype(jnp.uint16), jnp.bfloat16)
    c = lax.bitcast_convert_type(w.astype(jnp.uint16), jnp.bfloat16)
    return h, c.astype(jnp.float32)


def _update_body(g_ref, w_ref, b_ref, out_ref):
    h0c, c0c = _unpack_hc(g_ref[0])
    h1c, c1c = _unpack_hc(g_ref[1])
    z = (jnp.dot(h0c, w_ref[:H_SIZE, :],
                 preferred_element_type=jnp.float32)
         + jnp.dot(h1c, w_ref[H_SIZE:, :],
                   preferred_element_type=jnp.float32)
         + b_ref[...])
    f0 = jax.nn.sigmoid(z[:, :H_SIZE])
    f1 = jax.nn.sigmoid(z[:, H_SIZE:H2])
    i = jax.nn.sigmoid(z[:, H2:H2 + H_SIZE])
    o = jax.nn.sigmoid(z[:, H2 + H_SIZE:2 * H2])
    u = jnp.tanh(z[:, 2 * H2:])
    c = i * u + f0 * c0c + f1 * c1c
    out_ref[:, :H_SIZE] = o * jnp.tanh(c)
    out_ref[:, H_SIZE:] = c


def _update(g3, w_cat, b_cat):
    return pl.pallas_call(
        _update_body,
        grid=(M // _BM,),
        in_specs=[
            pl.BlockSpec((2, _BM, H_SIZE), lambda j: (0, j, 0)),
            pl.BlockSpec((H2, H2 + H3), lambda j: (0, 0)),
            pl.BlockSpec((1, H2 + H3), lambda j: (0, 0)),
        ],
        out_specs=pl.BlockSpec((_BM, H2), lambda j: (j, 0)),
        out_shape=jax.ShapeDtypeStruct((M, H2), jnp.float32),
    )(g3, w_cat, b_cat)


def kernel(x, child_idx, W_iou_w, W_iou_b, U_iou_w, U_iou_b, U_f_w, U_f_b):
    w_iou_t = W_iou_w.T
    b_iou = W_iou_b.reshape(1, H3)
    w_cat = jnp.concatenate([U_f_w.T, U_iou_w.T],
                            axis=1).astype(jnp.bfloat16)
    b_cat = jnp.concatenate([U_f_b, U_iou_b]).reshape(1, H2 + H3)

    hc = _leaf_update(x, w_iou_t, b_iou)

    pad = _MP - M
    idx0 = jnp.pad(child_idx[:, 0], (0, pad))
    idx1 = jnp.pad(child_idx[:, 1], (0, pad))
    idx_flat = jnp.concatenate([idx0, idx1]).astype(jnp.int32)

    g = _sc_gather_kernel()(hc, idx_flat)
    g3 = g.reshape(2, _MP, H_SIZE)

    return _update(g3, w_cat, b_cat)


# speedup vs baseline: 3.7874x; 1.3538x over previous
import functools

import jax
import jax.numpy as jnp
from jax import lax
from jax.experimental import pallas as pl
from jax.experimental.pallas import tpu as pltpu
from jax.experimental.pallas import tpu_sc as plsc

N, M, X_SIZE, H_SIZE = 100000, 50000, 128, 128
H2, H3 = 2 * H_SIZE, 3 * H_SIZE

_NC, _NS = 2, 16
_NW = _NC * _NS

_MP = 51200
_R = 2 * _MP
_R_PW = _R // _NW
_CH = 128
_NCHUNK = _R_PW // _CH
_NBUF = 5
_NGRP = _NCHUNK // _NBUF

_BN = 2000
_BM = 2000

_PREC = lax.Precision.HIGHEST


def _leaf_body(x_ref, w_ref, b_ref, out_ref):
    iou = jnp.dot(x_ref[...], w_ref[...], precision=_PREC,
                  preferred_element_type=jnp.float32) + b_ref[...]
    i = jax.nn.sigmoid(iou[:, :H_SIZE])
    o = jax.nn.sigmoid(iou[:, H_SIZE:H2])
    u = jnp.tanh(iou[:, H2:])
    c0 = i * u
    h0 = o * jnp.tanh(c0)
    hb = lax.bitcast_convert_type(h0.astype(jnp.bfloat16),
                                  jnp.uint16).astype(jnp.uint32)
    cb = lax.bitcast_convert_type(c0.astype(jnp.bfloat16),
                                  jnp.uint16).astype(jnp.uint32)
    out_ref[...] = lax.bitcast_convert_type((hb << 16) | cb, jnp.float32)


def _leaf_update(x, w_t, b):
    return pl.pallas_call(
        _leaf_body,
        grid=(N // _BN,),
        in_specs=[
            pl.BlockSpec((_BN, X_SIZE), lambda j: (j, 0)),
            pl.BlockSpec((X_SIZE, H3), lambda j: (0, 0)),
            pl.BlockSpec((1, H3), lambda j: (0, 0)),
        ],
        out_specs=pl.BlockSpec((_BN, H_SIZE), lambda j: (j, 0)),
        out_shape=jax.ShapeDtypeStruct((N, H_SIZE), jnp.float32),
    )(x, w_t, b)


def _sc_gather_body(table_hbm, idx_hbm, out_hbm, idx_v, rows_v, gsems, wsems):
    wid = lax.axis_index("s") * _NC + lax.axis_index("c")
    base = wid * _R_PW
    pltpu.sync_copy(idx_hbm.at[pl.ds(base, _R_PW)], idx_v)

    def fire(c, b):
        pltpu.async_copy(table_hbm.at[idx_v.at[pl.ds(c * _CH, _CH)]],
                         rows_v.at[b], gsems.at[b])

    def drain_write(c, b):
        pltpu.make_async_copy(table_hbm.at[idx_v.at[pl.ds(c * _CH, _CH)]],
                              rows_v.at[b], gsems.at[b]).wait()
        cp = pltpu.make_async_copy(rows_v.at[b],
                                   out_hbm.at[pl.ds(base + c * _CH, _CH)],
                                   wsems.at[b])
        cp.start()
        cp.wait()

    for b in range(_NBUF):
        fire(b, b)

    def group(j, carry):
        for b in range(_NBUF):
            c = j * _NBUF + b
            drain_write(c, b)
            fire(c + _NBUF, b)
        return carry

    lax.fori_loop(0, _NGRP - 1, group, 0)
    for b in range(_NBUF):
        drain_write((_NGRP - 1) * _NBUF + b, b)


@functools.cache
def _sc_gather_kernel():
    return pl.kernel(
        _sc_gather_body,
        out_type=jax.ShapeDtypeStruct((_R, H_SIZE), jnp.float32),
        mesh=plsc.VectorSubcoreMesh(core_axis_name="c", subcore_axis_name="s",
                                    num_cores=_NC, num_subcores=_NS),
        scratch_types=[
            pltpu.VMEM((_R_PW,), jnp.int32),
            pltpu.VMEM((_NBUF, _CH, H_SIZE), jnp.float32),
            pltpu.SemaphoreType.DMA((_NBUF,)),
            pltpu.SemaphoreType.DMA((_NBUF,)),
        ],
    )


def _unpack_hc(packed_f32):
    w = lax.bitcast_convert_type(packed_f32, jnp.uint32)
    h = lax.bitcast_convert_type((w >> 16).astype(jnp.uint16), jnp.bfloat16)
    c = lax.bitcast_convert_type(w.astype(jnp.uint16), jnp.bfloat16)
    return h, c.astype(jnp.float32)


def _update_body(g_ref, w_ref, b_ref, out_ref):
    h0c, c0c = _unpack_hc(g_ref[0])
    h1c, c1c = _unpack_hc(g_ref[1])
    z = (jnp.dot(h0c, w_ref[:H_SIZE, :],
                 preferred_element_type=jnp.float32)
         + jnp.dot(h1c, w_ref[H_SIZE:, :],
                   preferred_element_type=jnp.float32)
         + b_ref[...])
    f0 = jax.nn.sigmoid(z[:, :H_SIZE])
    f1 = jax.nn.sigmoid(z[:, H_SIZE:H2])
    i = jax.nn.sigmoid(z[:, H2:H2 + H_SIZE])
    o = jax.nn.sigmoid(z[:, H2 + H_SIZE:2 * H2])
    u = jnp.tanh(z[:, 2 * H2:])
    c = i * u + f0 * c0c + f1 * c1c
    out_ref[:, :H_SIZE] = o * jnp.tanh(c)
    out_ref[:, H_SIZE:] = c


def _update(g3, w_cat, b_cat):
    return pl.pallas_call(
        _update_body,
        grid=(M // _BM,),
        in_specs=[
            pl.BlockSpec((2, _BM, H_SIZE), lambda j: (0, j, 0)),
            pl.BlockSpec((H2, H2 + H3), lambda j: (0, 0)),
            pl.BlockSpec((1, H2 + H3), lambda j: (0, 0)),
        ],
        out_specs=pl.BlockSpec((_BM, H2), lambda j: (j, 0)),
        out_shape=jax.ShapeDtypeStruct((M, H2), jnp.float32),
    )(g3, w_cat, b_cat)


def kernel(x, child_idx, W_iou_w, W_iou_b, U_iou_w, U_iou_b, U_f_w, U_f_b):
    w_iou_t = W_iou_w.T
    b_iou = W_iou_b.reshape(1, H3)
    w_cat = jnp.concatenate([U_f_w.T, U_iou_w.T],
                            axis=1).astype(jnp.bfloat16)
    b_cat = jnp.concatenate([U_f_b, U_iou_b]).reshape(1, H2 + H3)

    hc = _leaf_update(x, w_iou_t, b_iou)

    pad = _MP - M
    pad_rows = jnp.arange(pad, dtype=child_idx.dtype)
    idx0 = jnp.concatenate([child_idx[:, 0], pad_rows])
    idx1 = jnp.concatenate([child_idx[:, 1], pad_rows])
    idx_flat = jnp.concatenate([idx0, idx1]).astype(jnp.int32)

    g = _sc_gather_kernel()(hc, idx_flat)
    g3 = g.reshape(2, _MP, H_SIZE)

    return _update(g3, w_cat, b_cat)


# speedup vs baseline: 5.2978x; 1.3988x over previous
import functools

import jax
import jax.numpy as jnp
from jax import lax
from jax.experimental import pallas as pl
from jax.experimental.pallas import tpu as pltpu
from jax.experimental.pallas import tpu_sc as plsc

N, M, X_SIZE, H_SIZE = 100000, 50000, 128, 128
H2, H3 = 2 * H_SIZE, 3 * H_SIZE

_NC, _NS = 2, 16
_NW = _NC * _NS

_MP = 51200
_R = 2 * _MP
_R_PW = _R // _NW
_CH = 128
_NCHUNK = _R_PW // _CH
_NBUF = 5
_NGRP = _NCHUNK // _NBUF

_BN = 2000
_BM = 2000

_PREC = lax.Precision.HIGHEST


def _leaf_body(x_ref, w_ref, b_ref, out_ref):
    iou = jnp.dot(x_ref[...].astype(jnp.bfloat16), w_ref[...],
                  preferred_element_type=jnp.float32) + b_ref[...]
    i = jax.nn.sigmoid(iou[:, :H_SIZE])
    o = jax.nn.sigmoid(iou[:, H_SIZE:H2])
    u = jnp.tanh(iou[:, H2:])
    c0 = i * u
    h0 = o * jnp.tanh(c0)
    hb = lax.bitcast_convert_type(h0.astype(jnp.bfloat16),
                                  jnp.uint16).astype(jnp.uint32)
    cb = lax.bitcast_convert_type(c0.astype(jnp.bfloat16),
                                  jnp.uint16).astype(jnp.uint32)
    out_ref[...] = lax.bitcast_convert_type((hb << 16) | cb, jnp.float32)


def _leaf_update(x, w_t, b):
    return pl.pallas_call(
        _leaf_body,
        grid=(N // _BN,),
        in_specs=[
            pl.BlockSpec((_BN, X_SIZE), lambda j: (j, 0)),
            pl.BlockSpec((X_SIZE, H3), lambda j: (0, 0)),
            pl.BlockSpec((1, H3), lambda j: (0, 0)),
        ],
        out_specs=pl.BlockSpec((_BN, H_SIZE), lambda j: (j, 0)),
        out_shape=jax.ShapeDtypeStruct((N, H_SIZE), jnp.float32),
    )(x, w_t, b)


def _sc_gather_body(table_hbm, idx_hbm, out_hbm, idx_v, rows_v, gsems, wsems):
    wid = lax.axis_index("s") * _NC + lax.axis_index("c")
    base = wid * _R_PW
    pltpu.sync_copy(idx_hbm.at[pl.ds(base, _R_PW)], idx_v)

    def fire(c, b):
        pltpu.async_copy(table_hbm.at[idx_v.at[pl.ds(c * _CH, _CH)]],
                         rows_v.at[b], gsems.at[b])

    def drain_write(c, b):
        pltpu.make_async_copy(table_hbm.at[idx_v.at[pl.ds(c * _CH, _CH)]],
                              rows_v.at[b], gsems.at[b]).wait()
        cp = pltpu.make_async_copy(rows_v.at[b],
                                   out_hbm.at[pl.ds(base + c * _CH, _CH)],
                                   wsems.at[b])
        cp.start()
        cp.wait()

    for b in range(_NBUF):
        fire(b, b)

    def group(j, carry):
        for b in range(_NBUF):
            c = j * _NBUF + b
            drain_write(c, b)
            fire(c + _NBUF, b)
        return carry

    lax.fori_loop(0, _NGRP - 1, group, 0)
    for b in range(_NBUF):
        drain_write((_NGRP - 1) * _NBUF + b, b)


@functools.cache
def _sc_gather_kernel():
    return pl.kernel(
        _sc_gather_body,
        out_type=jax.ShapeDtypeStruct((_R, H_SIZE), jnp.float32),
        mesh=plsc.VectorSubcoreMesh(core_axis_name="c", subcore_axis_name="s",
                                    num_cores=_NC, num_subcores=_NS),
        scratch_types=[
            pltpu.VMEM((_R_PW,), jnp.int32),
            pltpu.VMEM((_NBUF, _CH, H_SIZE), jnp.float32),
            pltpu.SemaphoreType.DMA((_NBUF,)),
            pltpu.SemaphoreType.DMA((_NBUF,)),
        ],
    )


def _unpack_hc(packed_f32):
    w = lax.bitcast_convert_type(packed_f32, jnp.uint32)
    h = lax.bitcast_convert_type((w >> 16).astype(jnp.uint16), jnp.bfloat16)
    c = lax.bitcast_convert_type(w.astype(jnp.uint16), jnp.bfloat16)
    return h, c.astype(jnp.float32)


def _update_body(g_ref, w_ref, b_ref, out_ref):
    h0c, c0c = _unpack_hc(g_ref[0])
    h1c, c1c = _unpack_hc(g_ref[1])
    z = (jnp.dot(h0c, w_ref[:H_SIZE, :],
                 preferred_element_type=jnp.float32)
         + jnp.dot(h1c, w_ref[H_SIZE:, :],
                   preferred_element_type=jnp.float32)
         + b_ref[...])
    f0 = jax.nn.sigmoid(z[:, :H_SIZE])
    f1 = jax.nn.sigmoid(z[:, H_SIZE:H2])
    i = jax.nn.sigmoid(z[:, H2:H2 + H_SIZE])
    o = jax.nn.sigmoid(z[:, H2 + H_SIZE:2 * H2])
    u = jnp.tanh(z[:, 2 * H2:])
    c = i * u + f0 * c0c + f1 * c1c
    out_ref[:, :H_SIZE] = o * jnp.tanh(c)
    out_ref[:, H_SIZE:] = c


def _update(g3, w_cat, b_cat):
    return pl.pallas_call(
        _update_body,
        grid=(M // _BM,),
        in_specs=[
            pl.BlockSpec((2, _BM, H_SIZE), lambda j: (0, j, 0)),
            pl.BlockSpec((H2, H2 + H3), lambda j: (0, 0)),
            pl.BlockSpec((1, H2 + H3), lambda j: (0, 0)),
        ],
        out_specs=pl.BlockSpec((_BM, H2), lambda j: (j, 0)),
        out_shape=jax.ShapeDtypeStruct((M, H2), jnp.float32),
    )(g3, w_cat, b_cat)


def kernel(x, child_idx, W_iou_w, W_iou_b, U_iou_w, U_iou_b, U_f_w, U_f_b):
    w_iou_t = W_iou_w.T.astype(jnp.bfloat16)
    b_iou = W_iou_b.reshape(1, H3)
    w_cat = jnp.concatenate([U_f_w.T, U_iou_w.T],
                            axis=1).astype(jnp.bfloat16)
    b_cat = jnp.concatenate([U_f_b, U_iou_b]).reshape(1, H2 + H3)

    hc = _leaf_update(x, w_iou_t, b_iou)

    pad = _MP - M
    pad_rows = jnp.arange(pad, dtype=child_idx.dtype)
    idx0 = jnp.concatenate([child_idx[:, 0], pad_rows])
    idx1 = jnp.concatenate([child_idx[:, 1], pad_rows])
    idx_flat = jnp.concatenate([idx0, idx1]).astype(jnp.int32)

    g = _sc_gather_kernel()(hc, idx_flat)
    g3 = g.reshape(2, _MP, H_SIZE)

    return _update(g3, w_cat, b_cat)


# speedup vs baseline: 5.8573x; 1.1056x over previous
import functools

import jax
import jax.numpy as jnp
from jax import lax
from jax.experimental import pallas as pl
from jax.experimental.pallas import tpu as pltpu
from jax.experimental.pallas import tpu_sc as plsc

N, M, X_SIZE, H_SIZE = 100000, 50000, 128, 128
H2, H3 = 2 * H_SIZE, 3 * H_SIZE

_NC, _NS = 2, 16
_NW = _NC * _NS

_MP = 51200
_R = 2 * _MP
_R_PW = _R // _NW
_CH = 128
_NCHUNK = _R_PW // _CH
_NBUF = 5
_NGRP = _NCHUNK // _NBUF

_BN = 4000
_BM = 5000

_PREC = lax.Precision.HIGHEST


def _leaf_body(x_ref, w_ref, b_ref, out_ref):
    iou = jnp.dot(x_ref[...].astype(jnp.bfloat16), w_ref[...],
                  preferred_element_type=jnp.float32) + b_ref[...]
    i = jax.nn.sigmoid(iou[:, :H_SIZE])
    o = jax.nn.sigmoid(iou[:, H_SIZE:H2])
    u = jnp.tanh(iou[:, H2:])
    c0 = i * u
    h0 = o * jnp.tanh(c0)
    hb = lax.bitcast_convert_type(h0.astype(jnp.bfloat16),
                                  jnp.uint16).astype(jnp.uint32)
    cb = lax.bitcast_convert_type(c0.astype(jnp.bfloat16),
                                  jnp.uint16).astype(jnp.uint32)
    out_ref[...] = lax.bitcast_convert_type((hb << 16) | cb, jnp.float32)


def _leaf_update(x, w_t, b):
    return pl.pallas_call(
        _leaf_body,
        grid=(N // _BN,),
        in_specs=[
            pl.BlockSpec((_BN, X_SIZE), lambda j: (j, 0)),
            pl.BlockSpec((X_SIZE, H3), lambda j: (0, 0)),
            pl.BlockSpec((1, H3), lambda j: (0, 0)),
        ],
        out_specs=pl.BlockSpec((_BN, H_SIZE), lambda j: (j, 0)),
        out_shape=jax.ShapeDtypeStruct((N, H_SIZE), jnp.float32),
    )(x, w_t, b)


def _sc_gather_body(table_hbm, idx_hbm, out_hbm, idx_v, rows_v, gsems, wsems):
    wid = lax.axis_index("s") * _NC + lax.axis_index("c")
    base = wid * _R_PW
    pltpu.sync_copy(idx_hbm.at[pl.ds(base, _R_PW)], idx_v)

    def fire(c, b):
        pltpu.async_copy(table_hbm.at[idx_v.at[pl.ds(c * _CH, _CH)]],
                         rows_v.at[b], gsems.at[b])

    def drain_write(c, b):
        pltpu.make_async_copy(table_hbm.at[idx_v.at[pl.ds(c * _CH, _CH)]],
                              rows_v.at[b], gsems.at[b]).wait()
        cp = pltpu.make_async_copy(rows_v.at[b],
                                   out_hbm.at[pl.ds(base + c * _CH, _CH)],
                                   wsems.at[b])
        cp.start()
        cp.wait()

    for b in range(_NBUF):
        fire(b, b)

    def group(j, carry):
        for b in range(_NBUF):
            c = j * _NBUF + b
            drain_write(c, b)
            fire(c + _NBUF, b)
        return carry

    lax.fori_loop(0, _NGRP - 1, group, 0)
    for b in range(_NBUF):
        drain_write((_NGRP - 1) * _NBUF + b, b)


@functools.cache
def _sc_gather_kernel():
    return pl.kernel(
        _sc_gather_body,
        out_type=jax.ShapeDtypeStruct((_R, H_SIZE), jnp.float32),
        mesh=plsc.VectorSubcoreMesh(core_axis_name="c", subcore_axis_name="s",
                                    num_cores=_NC, num_subcores=_NS),
        scratch_types=[
            pltpu.VMEM((_R_PW,), jnp.int32),
            pltpu.VMEM((_NBUF, _CH, H_SIZE), jnp.float32),
            pltpu.SemaphoreType.DMA((_NBUF,)),
            pltpu.SemaphoreType.DMA((_NBUF,)),
        ],
    )


def _unpack_hc(packed_f32):
    w = lax.bitcast_convert_type(packed_f32, jnp.uint32)
    h = lax.bitcast_convert_type((w >> 16).astype(jnp.uint16), jnp.bfloat16)
    c = lax.bitcast_convert_type(w.astype(jnp.uint16), jnp.bfloat16)
    return h, c.astype(jnp.float32)


def _update_body(g_ref, w_ref, b_ref, out_ref):
    h0c, c0c = _unpack_hc(g_ref[0])
    h1c, c1c = _unpack_hc(g_ref[1])
    z = (jnp.dot(h0c, w_ref[:H_SIZE, :],
                 preferred_element_type=jnp.float32)
         + jnp.dot(h1c, w_ref[H_SIZE:, :],
                   preferred_element_type=jnp.float32)
         + b_ref[...])
    f0 = jax.nn.sigmoid(z[:, :H_SIZE])
    f1 = jax.nn.sigmoid(z[:, H_SIZE:H2])
    i = jax.nn.sigmoid(z[:, H2:H2 + H_SIZE])
    o = jax.nn.sigmoid(z[:, H2 + H_SIZE:2 * H2])
    u = jnp.tanh(z[:, 2 * H2:])
    c = i * u + f0 * c0c + f1 * c1c
    out_ref[:, :H_SIZE] = o * jnp.tanh(c)
    out_ref[:, H_SIZE:] = c


def _update(g3, w_cat, b_cat):
    return pl.pallas_call(
        _update_body,
        grid=(M // _BM,),
        in_specs=[
            pl.BlockSpec((2, _BM, H_SIZE), lambda j: (0, j, 0)),
            pl.BlockSpec((H2, H2 + H3), lambda j: (0, 0)),
            pl.BlockSpec((1, H2 + H3), lambda j: (0, 0)),
        ],
        out_specs=pl.BlockSpec((_BM, H2), lambda j: (j, 0)),
        out_shape=jax.ShapeDtypeStruct((M, H2), jnp.float32),
    )(g3, w_cat, b_cat)


def kernel(x, child_idx, W_iou_w, W_iou_b, U_iou_w, U_iou_b, U_f_w, U_f_b):
    w_iou_t = W_iou_w.T.astype(jnp.bfloat16)
    b_iou = W_iou_b.reshape(1, H3)
    w_cat = jnp.concatenate([U_f_w.T, U_iou_w.T],
                            axis=1).astype(jnp.bfloat16)
    b_cat = jnp.concatenate([U_f_b, U_iou_b]).reshape(1, H2 + H3)

    hc = _leaf_update(x, w_iou_t, b_iou)

    pad = _MP - M
    pad_rows = jnp.arange(pad, dtype=child_idx.dtype)
    idx0 = jnp.concatenate([child_idx[:, 0], pad_rows])
    idx1 = jnp.concatenate([child_idx[:, 1], pad_rows])
    idx_flat = jnp.concatenate([idx0, idx1]).astype(jnp.int32)

    g = _sc_gather_kernel()(hc, idx_flat)
    g3 = g.reshape(2, _MP, H_SIZE)

    return _update(g3, w_cat, b_cat)


# speedup vs baseline: 6.1263x; 1.0459x over previous
import functools

import jax
import jax.numpy as jnp
from jax import lax
from jax.experimental import pallas as pl
from jax.experimental.pallas import tpu as pltpu
from jax.experimental.pallas import tpu_sc as plsc

N, M, X_SIZE, H_SIZE = 100000, 50000, 128, 128
H2, H3 = 2 * H_SIZE, 3 * H_SIZE

_NC, _NS = 2, 16
_NW = _NC * _NS

_MP = 51200
_R = 2 * _MP
_R_PW = _R // _NW
_CH = 128
_NCHUNK = _R_PW // _CH
_NBUF = 5
_NGRP = _NCHUNK // _NBUF

_BN = 10000
_BM = 5000

_PREC = lax.Precision.HIGHEST


def _leaf_body(x_ref, w_ref, b_ref, out_ref):
    iou = jnp.dot(x_ref[...].astype(jnp.bfloat16), w_ref[...],
                  preferred_element_type=jnp.float32) + b_ref[...]
    i = jax.nn.sigmoid(iou[:, :H_SIZE])
    o = jax.nn.sigmoid(iou[:, H_SIZE:H2])
    u = jnp.tanh(iou[:, H2:])
    c0 = i * u
    h0 = o * jnp.tanh(c0)
    hb = lax.bitcast_convert_type(h0.astype(jnp.bfloat16),
                                  jnp.uint16).astype(jnp.uint32)
    cb = lax.bitcast_convert_type(c0.astype(jnp.bfloat16),
                                  jnp.uint16).astype(jnp.uint32)
    out_ref[...] = lax.bitcast_convert_type((hb << 16) | cb, jnp.float32)


def _leaf_update(x, w_t, b):
    return pl.pallas_call(
        _leaf_body,
        grid=(N // _BN,),
        in_specs=[
            pl.BlockSpec((_BN, X_SIZE), lambda j: (j, 0)),
            pl.BlockSpec((X_SIZE, H3), lambda j: (0, 0)),
            pl.BlockSpec((1, H3), lambda j: (0, 0)),
        ],
        out_specs=pl.BlockSpec((_BN, H_SIZE), lambda j: (j, 0)),
        out_shape=jax.ShapeDtypeStruct((N, H_SIZE), jnp.float32),
    )(x, w_t, b)


def _sc_gather_body(table_hbm, idx_hbm, out_hbm, idx_v, rows_v, gsems, wsems):
    wid = lax.axis_index("s") * _NC + lax.axis_index("c")
    base = wid * _R_PW
    pltpu.sync_copy(idx_hbm.at[pl.ds(base, _R_PW)], idx_v)

    def fire(c, b):
        pltpu.async_copy(table_hbm.at[idx_v.at[pl.ds(c * _CH, _CH)]],
                         rows_v.at[b], gsems.at[b])

    def drain_write(c, b):
        pltpu.make_async_copy(table_hbm.at[idx_v.at[pl.ds(c * _CH, _CH)]],
                              rows_v.at[b], gsems.at[b]).wait()
        cp = pltpu.make_async_copy(rows_v.at[b],
                                   out_hbm.at[pl.ds(base + c * _CH, _CH)],
                                   wsems.at[b])
        cp.start()
        cp.wait()

    for b in range(_NBUF):
        fire(b, b)

    def group(j, carry):
        for b in range(_NBUF):
            c = j * _NBUF + b
            drain_write(c, b)
            fire(c + _NBUF, b)
        return carry

    lax.fori_loop(0, _NGRP - 1, group, 0)
    for b in range(_NBUF):
        drain_write((_NGRP - 1) * _NBUF + b, b)


@functools.cache
def _sc_gather_kernel():
    return pl.kernel(
        _sc_gather_body,
        out_type=jax.ShapeDtypeStruct((_R, H_SIZE), jnp.float32),
        mesh=plsc.VectorSubcoreMesh(core_axis_name="c", subcore_axis_name="s",
                                    num_cores=_NC, num_subcores=_NS),
        scratch_types=[
            pltpu.VMEM((_R_PW,), jnp.int32),
            pltpu.VMEM((_NBUF, _CH, H_SIZE), jnp.float32),
            pltpu.SemaphoreType.DMA((_NBUF,)),
            pltpu.SemaphoreType.DMA((_NBUF,)),
        ],
    )


def _unpack_hc(packed_f32):
    w = lax.bitcast_convert_type(packed_f32, jnp.uint32)
    h = lax.bitcast_convert_type((w >> 16).astype(jnp.uint16), jnp.bfloat16)
    c = lax.bitcast_convert_type(w.astype(jnp.uint16), jnp.bfloat16)
    return h, c.astype(jnp.float32)


def _update_body(g_ref, w_ref, b_ref, out_ref):
    h0c, c0c = _unpack_hc(g_ref[0])
    h1c, c1c = _unpack_hc(g_ref[1])
    z = (jnp.dot(h0c, w_ref[:H_SIZE, :],
                 preferred_element_type=jnp.float32)
         + jnp.dot(h1c, w_ref[H_SIZE:, :],
                   preferred_element_type=jnp.float32)
         + b_ref[...])
    f0 = jax.nn.sigmoid(z[:, :H_SIZE])
    f1 = jax.nn.sigmoid(z[:, H_SIZE:H2])
    i = jax.nn.sigmoid(z[:, H2:H2 + H_SIZE])
    o = jax.nn.sigmoid(z[:, H2 + H_SIZE:2 * H2])
    u = jnp.tanh(z[:, 2 * H2:])
    c = i * u + f0 * c0c + f1 * c1c
    out_ref[:, :H_SIZE] = o * jnp.tanh(c)
    out_ref[:, H_SIZE:] = c


def _update(g3, w_cat, b_cat):
    return pl.pallas_call(
        _update_body,
        grid=(M // _BM,),
        in_specs=[
            pl.BlockSpec((2, _BM, H_SIZE), lambda j: (0, j, 0)),
            pl.BlockSpec((H2, H2 + H3), lambda j: (0, 0)),
            pl.BlockSpec((1, H2 + H3), lambda j: (0, 0)),
        ],
        out_specs=pl.BlockSpec((_BM, H2), lambda j: (j, 0)),
        out_shape=jax.ShapeDtypeStruct((M, H2), jnp.float32),
    )(g3, w_cat, b_cat)


def kernel(x, child_idx, W_iou_w, W_iou_b, U_iou_w, U_iou_b, U_f_w, U_f_b):
    w_iou_t = W_iou_w.T.astype(jnp.bfloat16)
    b_iou = W_iou_b.reshape(1, H3)
    w_cat = jnp.concatenate([U_f_w.T, U_iou_w.T],
                            axis=1).astype(jnp.bfloat16)
    b_cat = jnp.concatenate([U_f_b, U_iou_b]).reshape(1, H2 + H3)

    hc = _leaf_update(x, w_iou_t, b_iou)

    pad = _MP - M
    pad_rows = jnp.arange(pad, dtype=child_idx.dtype)
    idx0 = jnp.concatenate([child_idx[:, 0], pad_rows])
    idx1 = jnp.concatenate([child_idx[:, 1], pad_rows])
    idx_flat = jnp.concatenate([idx0, idx1]).astype(jnp.int32)

    g = _sc_gather_kernel()(hc, idx_flat)
    g3 = g.reshape(2, _MP, H_SIZE)

    return _update(g3, w_cat, b_cat)
